# baseline (speedup 1.0000x reference)
import functools

import jax
import jax.numpy as jnp
from jax import lax
from jax.experimental import pallas as pl
from jax.experimental.pallas import tpu as pltpu
from jax.experimental.pallas import tpu_sc as plsc

_PERIODS = (64, 256, 1024, 8192)
_D = 2048
_Q = 8192
_BQ = 256
_K = 8
_NB = 4


def _shifted_rows(ref, start, nrows):
    aligned = pl.multiple_of((start // 8) * 8, 8)
    r = start % 8
    n = nrows + 8
    chunk = ref[pl.ds(aligned, n), :]
    rolled = pltpu.roll(chunk, (n - r) % n, axis=0)
    return rolled[:nrows, :]


def _combine_body(s_ref, w0d_ref, w1d_ref, w2d_ref, w3_ref, u_ref,
                  r0_ref, r1_ref, r2_ref):
    b = pl.program_id(0)

    @pl.when(b == 0)
    def _():
        r0_ref[...] = _shifted_rows(w0d_ref, s_ref[0], _PERIODS[0])
        r1_ref[...] = _shifted_rows(w1d_ref, s_ref[1], _PERIODS[1])
        r2_ref[...] = _shifted_rows(w2d_ref, s_ref[2], _PERIODS[2])

    st2 = pl.multiple_of((b % (_PERIODS[2] // _BQ)) * _BQ, _BQ)
    acc = w3_ref[...] + r2_ref[pl.ds(st2, _BQ), :]
    acc = acc + jnp.concatenate([r1_ref[...]] * (_BQ // _PERIODS[1]), axis=0)
    acc = acc + jnp.concatenate([r0_ref[...]] * (_BQ // _PERIODS[0]), axis=0)
    u_ref[...] = acc


def _build_u(shifts, W0d, W1d, W2d, W3):
    return pl.pallas_call(
        _combine_body,
        grid=(_Q // _BQ,),
        in_specs=[
            pl.BlockSpec(memory_space=pltpu.SMEM),
            pl.BlockSpec((2 * _PERIODS[0], _D), lambda b: (0, 0)),
            pl.BlockSpec((2 * _PERIODS[1], _D), lambda b: (0, 0)),
            pl.BlockSpec((2 * _PERIODS[2], _D), lambda b: (0, 0)),
            pl.BlockSpec((_BQ, _D), lambda b: (b, 0)),
        ],
        out_specs=pl.BlockSpec((_BQ, _D), lambda b: (b, 0)),
        out_shape=jax.ShapeDtypeStruct((_Q, _D), jnp.float32),
        scratch_shapes=[
            pltpu.VMEM((_PERIODS[0], _D), jnp.float32),
            pltpu.VMEM((_PERIODS[1], _D), jnp.float32),
            pltpu.VMEM((_PERIODS[2], _D), jnp.float32),
        ],
    )(shifts, W0d, W1d, W2d, W3)


def _gather_rows(U, idx, n_rows, b_per_w):
    n_chunks = b_per_w // _K

    mesh = plsc.VectorSubcoreMesh(core_axis_name="c", subcore_axis_name="s")

    @functools.partial(
        pl.kernel,
        mesh=mesh,
        out_type=jax.ShapeDtypeStruct((n_rows, _D), jnp.float32),
        scratch_types=[
            pltpu.VMEM((b_per_w,), jnp.int32),
            pltpu.VMEM((_NB, _K, _D), jnp.float32),
            pltpu.SemaphoreType.DMA((_NB,)),
            pltpu.SemaphoreType.DMA((_NB,)),
        ],
    )
    def gather_kernel(u_hbm, idx_hbm, out_hbm, idx_v, rows_v, gsem, wsem):
        n_cores = lax.axis_size("c")
        wid = lax.axis_index("s") * n_cores + lax.axis_index("c")
        base = wid * b_per_w
        pltpu.sync_copy(idx_hbm.at[pl.ds(base, b_per_w)], idx_v)

        last = n_chunks - 1

        def start_gather(c, buf):
            pltpu.async_copy(
                u_hbm.at[idx_v.at[pl.ds(c * _K, _K)]],
                rows_v.at[buf], gsem.at[buf])

        def wait_gather(buf):
            pltpu.make_async_copy(
                u_hbm.at[idx_v.at[pl.ds(0, _K)]],
                rows_v.at[buf], gsem.at[buf]).wait()

        def start_write(c, buf):
            pltpu.async_copy(
                rows_v.at[buf], out_hbm.at[pl.ds(base + c * _K, _K)],
                wsem.at[buf])

        def wait_write(buf):
            pltpu.make_async_copy(
                rows_v.at[buf], out_hbm.at[pl.ds(base, _K)],
                wsem.at[buf]).wait()

        start_gather(0, 0)
        start_gather(1, 1)
        for j in range(_NB):
            wait_gather(j)
            start_write(j, j)
            bn = (j + 2) % _NB
            if j >= 2:
                wait_write(bn)
            start_gather(j + 2, bn)

        def body(i, _):
            for j in range(_NB):
                c = i * _NB + j
                wait_gather(j)
                start_write(c, j)
                bn = (j + 2) % _NB
                wait_write(bn)
                start_gather(jnp.minimum(c + 2, last), bn)
            return 0

        lax.fori_loop(1, n_chunks // _NB, body, 0)

        wait_gather(0)
        wait_gather(1)
        wait_write(2)
        wait_write(3)

    return gather_kernel(U, idx)


def kernel(pos, offsets, W0, W1, W2, W3):
    periods = jnp.asarray(_PERIODS, dtype=jnp.int32)
    shifts = (offsets - offsets[3]) % periods
    W0d = jnp.concatenate([W0, W0], axis=0)
    W1d = jnp.concatenate([W1, W1], axis=0)
    W2d = jnp.concatenate([W2, W2], axis=0)
    U = _build_u(shifts, W0d, W1d, W2d, W3)

    n_rows = pos.shape[0] * pos.shape[1]
    idx = (pos.reshape(n_rows) + offsets[3]) % _Q
    b_per_w = n_rows // 32
    out = _gather_rows(U, idx.astype(jnp.int32), n_rows, b_per_w)
    return out.reshape(pos.shape[0], pos.shape[1], _D)


# speedup vs baseline: 1.0951x; 1.0951x over previous
import functools

import jax
import jax.numpy as jnp
from jax import lax
from jax.experimental import pallas as pl
from jax.experimental.pallas import tpu as pltpu
from jax.experimental.pallas import tpu_sc as plsc

_PERIODS = (64, 256, 1024, 8192)
_D = 2048
_Q = 8192
_BQ = 512
_K = 8
_NB = 4


def _shifted_table(ref, shift, period):
    return pltpu.roll(ref[...], period - shift, axis=0)


def _combine_body(s_ref, w0_ref, w1_ref, w2_ref, w3_ref, u_ref,
                  r0_ref, r1_ref, r2_ref):
    b = pl.program_id(0)

    @pl.when(b == 0)
    def _():
        r0_ref[...] = _shifted_table(w0_ref, s_ref[0], _PERIODS[0])
        r1_ref[...] = _shifted_table(w1_ref, s_ref[1], _PERIODS[1])
        r2_ref[...] = _shifted_table(w2_ref, s_ref[2], _PERIODS[2])

    st2 = pl.multiple_of((b % (_PERIODS[2] // _BQ)) * _BQ, _BQ)
    acc = w3_ref[...] + r2_ref[pl.ds(st2, _BQ), :]
    acc = acc + jnp.concatenate([r1_ref[...]] * (_BQ // _PERIODS[1]), axis=0)
    acc = acc + jnp.concatenate([r0_ref[...]] * (_BQ // _PERIODS[0]), axis=0)
    u_ref[...] = acc


def _build_u(shifts, W0, W1, W2, W3):
    return pl.pallas_call(
        _combine_body,
        grid=(_Q // _BQ,),
        in_specs=[
            pl.BlockSpec(memory_space=pltpu.SMEM),
            pl.BlockSpec((_PERIODS[0], _D), lambda b: (0, 0)),
            pl.BlockSpec((_PERIODS[1], _D), lambda b: (0, 0)),
            pl.BlockSpec((_PERIODS[2], _D), lambda b: (0, 0)),
            pl.BlockSpec((_BQ, _D), lambda b: (b, 0)),
        ],
        out_specs=pl.BlockSpec((_BQ, _D), lambda b: (b, 0)),
        out_shape=jax.ShapeDtypeStruct((_Q, _D), jnp.float32),
        scratch_shapes=[
            pltpu.VMEM((_PERIODS[0], _D), jnp.float32),
            pltpu.VMEM((_PERIODS[1], _D), jnp.float32),
            pltpu.VMEM((_PERIODS[2], _D), jnp.float32),
        ],
    )(shifts, W0, W1, W2, W3)


def _gather_rows(U, idx, n_rows, b_per_w):
    n_chunks = b_per_w // _K

    mesh = plsc.VectorSubcoreMesh(core_axis_name="c", subcore_axis_name="s")

    @functools.partial(
        pl.kernel,
        mesh=mesh,
        out_type=jax.ShapeDtypeStruct((n_rows, _D), jnp.float32),
        scratch_types=[
            pltpu.VMEM((b_per_w,), jnp.int32),
            pltpu.VMEM((_NB, _K, _D), jnp.float32),
            pltpu.SemaphoreType.DMA((_NB,)),
            pltpu.SemaphoreType.DMA((_NB,)),
        ],
    )
    def gather_kernel(u_hbm, idx_hbm, out_hbm, idx_v, rows_v, gsem, wsem):
        n_cores = lax.axis_size("c")
        wid = lax.axis_index("s") * n_cores + lax.axis_index("c")
        base = wid * b_per_w
        pltpu.sync_copy(idx_hbm.at[pl.ds(base, b_per_w)], idx_v)

        last = n_chunks - 1

        def start_gather(c, buf):
            pltpu.async_copy(
                u_hbm.at[idx_v.at[pl.ds(c * _K, _K)]],
                rows_v.at[buf], gsem.at[buf])

        def wait_gather(buf):
            pltpu.make_async_copy(
                u_hbm.at[idx_v.at[pl.ds(0, _K)]],
                rows_v.at[buf], gsem.at[buf]).wait()

        def start_write(c, buf):
            pltpu.async_copy(
                rows_v.at[buf], out_hbm.at[pl.ds(base + c * _K, _K)],
                wsem.at[buf])

        def wait_write(buf):
            pltpu.make_async_copy(
                rows_v.at[buf], out_hbm.at[pl.ds(base, _K)],
                wsem.at[buf]).wait()

        start_gather(0, 0)
        start_gather(1, 1)
        for j in range(_NB):
            wait_gather(j)
            start_write(j, j)
            bn = (j + 2) % _NB
            if j >= 2:
                wait_write(bn)
            start_gather(j + 2, bn)

        def body(i, _):
            for j in range(_NB):
                c = i * _NB + j
                wait_gather(j)
                start_write(c, j)
                bn = (j + 2) % _NB
                wait_write(bn)
                start_gather(jnp.minimum(c + 2, last), bn)
            return 0

        lax.fori_loop(1, n_chunks // _NB, body, 0)

        wait_gather(0)
        wait_gather(1)
        wait_write(2)
        wait_write(3)

    return gather_kernel(U, idx)


def kernel(pos, offsets, W0, W1, W2, W3):
    periods = jnp.asarray(_PERIODS, dtype=jnp.int32)
    shifts = (offsets - offsets[3]) % periods
    U = _build_u(shifts, W0, W1, W2, W3)

    n_rows = pos.shape[0] * pos.shape[1]
    idx = (pos.reshape(n_rows) + offsets[3]) % _Q
    b_per_w = n_rows // 32
    out = _gather_rows(U, idx.astype(jnp.int32), n_rows, b_per_w)
    return out.reshape(pos.shape[0], pos.shape[1], _D)
